# Initial kernel scaffold; baseline (speedup 1.0000x reference)
import functools

import jax
import jax.numpy as jnp
from jax import lax
from jax.experimental import pallas as pl
from jax.experimental.pallas import tpu as pltpu
from jax.experimental.pallas import tpu_sc as plsc

NUM_ROWS = 16384 * 50
DIM = 64
CHUNK = 128


def _build(num_workers: int, n_chunks: int):
  b_per_w = NUM_ROWS // num_workers
  mesh = plsc.VectorSubcoreMesh(core_axis_name="c", subcore_axis_name="s")
  nc = mesh.num_cores

  @functools.partial(
      pl.kernel,
      out_type=jax.ShapeDtypeStruct((NUM_ROWS, DIM), jnp.float32),
      mesh=mesh,
      scratch_types=[
          pltpu.VMEM((n_chunks, CHUNK), jnp.int32),
          pltpu.VMEM((CHUNK, DIM), jnp.float32),
          pltpu.SemaphoreType.DMA,
      ],
  )
  def gather_kernel(ids_hbm, table_hbm, out_hbm, idx_v, rows_v, sem):
    wid = lax.axis_index("s") * nc + lax.axis_index("c")
    base = wid * b_per_w
    pltpu.sync_copy(ids_hbm.at[wid], idx_v)

    @pl.loop(0, n_chunks)
    def _(j):
      pltpu.async_copy(table_hbm.at[idx_v.at[j]], rows_v, sem).wait()
      pltpu.sync_copy(rows_v, out_hbm.at[pl.ds(base + j * CHUNK, CHUNK)])

  return gather_kernel


def kernel(ids, table):
  num_workers = 32
  n_chunks = NUM_ROWS // num_workers // CHUNK
  ids_flat = ids.reshape(-1).astype(jnp.int32)
  ids_grp = ids_flat.reshape(num_workers, n_chunks, CHUNK)
  out = _build(num_workers, n_chunks)(ids_grp, table)
  return out.reshape(ids.shape[0], ids.shape[1], DIM)


# speedup vs baseline: 1.6860x; 1.6860x over previous
import functools

import jax
import jax.numpy as jnp
from jax import lax
from jax.experimental import pallas as pl
from jax.experimental.pallas import tpu as pltpu
from jax.experimental.pallas import tpu_sc as plsc

NUM_ROWS = 16384 * 50
DIM = 64
CHUNK = 128


def _build(num_workers: int, n_chunks: int):
  b_per_w = NUM_ROWS // num_workers
  mesh = plsc.VectorSubcoreMesh(core_axis_name="c", subcore_axis_name="s")
  nc = mesh.num_cores

  @functools.partial(
      pl.kernel,
      out_type=jax.ShapeDtypeStruct((NUM_ROWS, DIM), jnp.float32),
      mesh=mesh,
      scratch_types=[
          pltpu.VMEM((n_chunks, CHUNK), jnp.int32),
          pltpu.VMEM((CHUNK, DIM), jnp.float32),
          pltpu.SemaphoreType.DMA,
      ],
      compiler_params=pltpu.CompilerParams(use_tc_tiling_on_sc=False),
  )
  def gather_kernel(ids_hbm, table_hbm, out_hbm, idx_v, rows_v, sem):
    wid = lax.axis_index("s") * nc + lax.axis_index("c")
    base = wid * b_per_w
    pltpu.sync_copy(ids_hbm.at[wid], idx_v)

    @pl.loop(0, n_chunks)
    def _(j):
      pltpu.async_copy(table_hbm.at[idx_v.at[j]], rows_v, sem).wait()
      pltpu.sync_copy(rows_v, out_hbm.at[pl.ds(base + j * CHUNK, CHUNK)])

  return gather_kernel


def kernel(ids, table):
  num_workers = 32
  n_chunks = NUM_ROWS // num_workers // CHUNK
  ids_flat = ids.reshape(-1).astype(jnp.int32)
  ids_grp = ids_flat.reshape(num_workers, n_chunks, CHUNK)
  out = _build(num_workers, n_chunks)(ids_grp, table)
  return out.reshape(ids.shape[0], ids.shape[1], DIM)


# speedup vs baseline: 1.7500x; 1.0380x over previous
import functools

import jax
import jax.numpy as jnp
from jax import lax
from jax.experimental import pallas as pl
from jax.experimental.pallas import tpu as pltpu
from jax.experimental.pallas import tpu_sc as plsc

NUM_ROWS = 16384 * 50
DIM = 64
CHUNK = 128


def _build(num_workers: int, n_chunks: int):
  b_per_w = NUM_ROWS // num_workers
  mesh = plsc.VectorSubcoreMesh(core_axis_name="c", subcore_axis_name="s")
  nc = mesh.num_cores

  @functools.partial(
      pl.kernel,
      out_type=jax.ShapeDtypeStruct((NUM_ROWS, DIM), jnp.float32),
      mesh=mesh,
      scratch_types=[
          pltpu.VMEM((n_chunks, CHUNK), jnp.int32),
          pltpu.VMEM((2, CHUNK, DIM), jnp.float32),
          pltpu.SemaphoreType.DMA,
      ],
      compiler_params=pltpu.CompilerParams(use_tc_tiling_on_sc=False),
  )
  def gather_kernel(ids_hbm, table_hbm, out_hbm, idx_v, rows_v, sem):
    wid = lax.axis_index("s") * nc + lax.axis_index("c")
    base = wid * b_per_w
    pltpu.sync_copy(ids_hbm.at[wid], idx_v)

    def fire(g, slot):
      pltpu.async_copy(table_hbm.at[idx_v.at[g]], rows_v.at[slot], sem)

    def drain(slot):
      pltpu.make_async_copy(
          table_hbm.at[idx_v.at[0]], rows_v.at[slot], sem
      ).wait()

    fire(0, 0)

    @pl.loop(0, n_chunks, step=2)
    def _(j):
      for b in range(2):
        g = j + b
        drain(b)

        @pl.when(g + 1 < n_chunks)
        def _():
          fire(g + 1, 1 - b)

        pltpu.sync_copy(
            rows_v.at[b], out_hbm.at[pl.ds(base + g * CHUNK, CHUNK)]
        )

  return gather_kernel


def kernel(ids, table):
  num_workers = 32
  n_chunks = NUM_ROWS // num_workers // CHUNK
  ids_flat = ids.reshape(-1).astype(jnp.int32)
  ids_grp = ids_flat.reshape(num_workers, n_chunks, CHUNK)
  out = _build(num_workers, n_chunks)(ids_grp, table)
  return out.reshape(ids.shape[0], ids.shape[1], DIM)


# speedup vs baseline: 1.8764x; 1.0722x over previous
import functools

import jax
import jax.numpy as jnp
from jax import lax
from jax.experimental import pallas as pl
from jax.experimental.pallas import tpu as pltpu
from jax.experimental.pallas import tpu_sc as plsc

NUM_ROWS = 16384 * 50
DIM = 64
CHUNK = 128
NBUF = 8


def _build(num_workers: int, n_chunks: int):
  b_per_w = NUM_ROWS // num_workers
  mesh = plsc.VectorSubcoreMesh(core_axis_name="c", subcore_axis_name="s")
  nc = mesh.num_cores

  @functools.partial(
      pl.kernel,
      out_type=jax.ShapeDtypeStruct((NUM_ROWS, DIM), jnp.float32),
      mesh=mesh,
      scratch_types=[
          pltpu.VMEM((n_chunks, CHUNK), jnp.int32),
          pltpu.VMEM((NBUF, CHUNK, DIM), jnp.float32),
          pltpu.SemaphoreType.DMA,
      ],
      compiler_params=pltpu.CompilerParams(use_tc_tiling_on_sc=False),
  )
  def gather_kernel(ids_hbm, table_hbm, out_hbm, idx_v, rows_v, sem):
    wid = lax.axis_index("s") * nc + lax.axis_index("c")
    base = wid * b_per_w
    pltpu.sync_copy(ids_hbm.at[wid], idx_v)

    def fire(g, slot):
      pltpu.async_copy(table_hbm.at[idx_v.at[g]], rows_v.at[slot], sem)

    def drain(slot):
      pltpu.make_async_copy(
          table_hbm.at[idx_v.at[0]], rows_v.at[slot], sem
      ).wait()

    for b in range(NBUF):
      fire(b, b)

    @pl.loop(0, n_chunks, step=NBUF)
    def _(j):
      for b in range(NBUF):
        g = j + b
        drain(b)
        pltpu.sync_copy(
            rows_v.at[b], out_hbm.at[pl.ds(base + g * CHUNK, CHUNK)]
        )

        @pl.when(g + NBUF < n_chunks)
        def _():
          fire(g + NBUF, b)

  return gather_kernel


def kernel(ids, table):
  num_workers = 32
  n_chunks = NUM_ROWS // num_workers // CHUNK
  ids_flat = ids.reshape(-1).astype(jnp.int32)
  ids_grp = ids_flat.reshape(num_workers, n_chunks, CHUNK)
  out = _build(num_workers, n_chunks)(ids_grp, table)
  return out.reshape(ids.shape[0], ids.shape[1], DIM)
